# Initial kernel scaffold; baseline (speedup 1.0000x reference)
import jax
import jax.numpy as jnp
from jax.experimental import pallas as pl


def kernel(ego_RWPE, glo_RWPE, x_emb, subgraphs_batch, batch, W_ego, b_ego, W_glo, b_glo, W_merge, b_merge, norm_w, norm_b):
    raise NotImplementedError("write your pallas kernel here")


# speedup vs baseline: 5.5792x; 5.5792x over previous
import functools

import jax
import jax.numpy as jnp
from jax import lax
from jax.experimental import pallas as pl
from jax.experimental.pallas import tpu as pltpu
from jax.experimental.pallas import tpu_sc as plsc

_M = 3_200_000
_N = 100_000
_G = 64
_NHID = 128
_EPS = 1e-5

_NC, _NS = 2, 16
_NW = _NC * _NS
_RW = _M // _NW
_B = 800
_C = 100
_NBLK = _RW // _B
_NCHUNK = _B // _C
_STR = 6256
_NP = _STR * _NS

_BN = 2000
_NBN = _N // _BN


def _sc_segment_sum(ego, ids2d, ones_c, z16, z1):
    mesh = plsc.VectorSubcoreMesh(core_axis_name="c", subcore_axis_name="s",
                                  num_cores=_NC, num_subcores=_NS)

    @functools.partial(
        pl.kernel,
        out_type=(
            jax.ShapeDtypeStruct((_NC, _NP, 16), jnp.float32),
            jax.ShapeDtypeStruct((_NC * _NP,), jnp.float32),
        ),
        mesh=mesh,
        compiler_params=pltpu.CompilerParams(use_tc_tiling_on_sc=False),
        scratch_types=[
            pltpu.VMEM((_B, 16), jnp.float32),
            pltpu.VMEM((_NCHUNK, _C), jnp.int32),
            pltpu.VMEM((_C,), jnp.float32),
            pltpu.VMEM((_STR,), jnp.float32),
            pltpu.VMEM_SHARED((_NP, 16), jnp.float32),
            pltpu.VMEM_SHARED((_NP,), jnp.float32),
            pltpu.SemaphoreType.DMA,
        ],
    )
    def k(ego_hbm, ids_hbm, ones_hbm, z16_hbm, z1_hbm, out16, outc,
          rowbuf, idxbuf, onesbuf, zb1, acc, accc, sem):
        c = lax.axis_index("c")
        s = lax.axis_index("s")
        wid = s * _NC + c
        pltpu.sync_copy(z16_hbm, rowbuf)
        pltpu.sync_copy(z1_hbm, zb1)
        for k in range(_STR // _B):
            pltpu.sync_copy(rowbuf, acc.at[pl.ds(s * _STR + k * _B, _B)])
        _REM = _STR - (_STR // _B) * _B
        if _REM:
            pltpu.sync_copy(rowbuf.at[pl.ds(0, _REM)],
                            acc.at[pl.ds(s * _STR + _STR - _REM, _REM)])
        pltpu.sync_copy(zb1, accc.at[pl.ds(s * _STR, _STR)])
        pltpu.sync_copy(ones_hbm, onesbuf)
        plsc.subcore_barrier()

        base = wid * _RW
        gbase = wid * (_RW // _C)

        def body(b, carry):
            pltpu.sync_copy(ego_hbm.at[pl.ds(base + b * _B, _B)], rowbuf)
            pltpu.sync_copy(ids_hbm.at[pl.ds(gbase + b * _NCHUNK, _NCHUNK)],
                            idxbuf)
            descs = []
            for j in range(_NCHUNK):
                descs.append(pltpu.async_copy(
                    rowbuf.at[pl.ds(j * _C, _C)],
                    acc.at[idxbuf.at[j]], sem, add=True))
                descs.append(pltpu.async_copy(
                    onesbuf, accc.at[idxbuf.at[j]], sem, add=True))
            for d in descs:
                d.wait()
            return carry

        lax.fori_loop(0, _NBLK, body, 0)
        plsc.subcore_barrier()
        for k in range(_STR // _B):
            pltpu.sync_copy(acc.at[pl.ds(s * _STR + k * _B, _B)], rowbuf)
            pltpu.sync_copy(rowbuf, out16.at[c, pl.ds(s * _STR + k * _B, _B)])
        if _REM:
            pltpu.sync_copy(acc.at[pl.ds(s * _STR + _STR - _REM, _REM)],
                            rowbuf.at[pl.ds(0, _REM)])
            pltpu.sync_copy(rowbuf.at[pl.ds(0, _REM)],
                            out16.at[c, pl.ds(s * _STR + _STR - _REM, _REM)])
        pltpu.sync_copy(accc.at[pl.ds(s * _STR, _STR)], zb1)
        pltpu.sync_copy(zb1, outc.at[pl.ds(c * _NP + s * _STR, _STR)])

    return k(ego, ids2d, ones_c, z16, z1)


def _merged_x(xe_ref, p_ref, c_ref, gl_ref, wx_ref, wp_ref, wg_ref,
              wc_ref, bias_ref):
    p16 = p_ref[0] + p_ref[1]
    cnt = c_ref[0, :, 0] + c_ref[1, :, 0]
    x = jnp.dot(xe_ref[...], wx_ref[...], preferred_element_type=jnp.float32)
    x += jnp.dot(p16, wp_ref[...], preferred_element_type=jnp.float32)
    x += jnp.dot(gl_ref[...], wg_ref[...], preferred_element_type=jnp.float32)
    x += cnt[:, None] * wc_ref[...]
    x += bias_ref[...]
    return x


def _onehot(b_ref):
    bb = b_ref[...]
    gi = lax.broadcasted_iota(jnp.int32, (_BN, _G), 1)
    return (bb == gi).astype(jnp.float32)


def _pass_a_body(xe_ref, p_ref, c_ref, gl_ref, b_ref, wx_ref, wp_ref, wg_ref,
                 wc_ref, bias_ref, stats_ref):
    x = _merged_x(xe_ref, p_ref, c_ref, gl_ref, wx_ref, wp_ref, wg_ref,
                  wc_ref, bias_ref)
    rs = jnp.sum(x, axis=1, keepdims=True)
    rq = jnp.sum(x * x, axis=1, keepdims=True)
    ones = jnp.ones_like(rs)
    rsq = jnp.concatenate([rs, rq, ones], axis=1)
    oh = _onehot(b_ref)
    part = lax.dot_general(oh, rsq, (((0,), (0,)), ((), ())),
                           preferred_element_type=jnp.float32)

    @pl.when(pl.program_id(0) == 0)
    def _():
        stats_ref[...] = jnp.zeros_like(stats_ref)

    stats_ref[...] += part


def _pass_b_body(xe_ref, p_ref, c_ref, gl_ref, b_ref, wx_ref, wp_ref, wg_ref,
                 wc_ref, bias_ref, stats_ref, nw_ref, nb_ref, out_ref):
    x = _merged_x(xe_ref, p_ref, c_ref, gl_ref, wx_ref, wp_ref, wg_ref,
                  wc_ref, bias_ref)
    st = stats_ref[...]
    cg = jnp.maximum(st[:, 2:3] * jnp.float32(_NHID), 1.0)
    mean = st[:, 0:1] / cg
    var = st[:, 1:2] / cg - mean * mean
    inv = lax.rsqrt(var + _EPS)
    m2 = jnp.concatenate([mean, inv], axis=1)
    oh = _onehot(b_ref)
    mi = jnp.dot(oh, m2, preferred_element_type=jnp.float32)
    out_ref[...] = ((x - mi[:, 0:1]) * mi[:, 1:2] * nw_ref[...]
                    + nb_ref[...])


def kernel(ego_RWPE, glo_RWPE, x_emb, subgraphs_batch, batch,
           W_ego, b_ego, W_glo, b_glo, W_merge, b_merge, norm_w, norm_b):
    ids = subgraphs_batch.astype(jnp.int32).reshape(_M // _C, _C)
    bat = batch.astype(jnp.int32).reshape(_N, 1)

    parts, cnts = _sc_segment_sum(
        ego_RWPE, ids,
        jnp.ones((_C,), jnp.float32),
        jnp.zeros((_B, 16), jnp.float32),
        jnp.zeros((_STR,), jnp.float32),
    )

    cnts3 = cnts.reshape(_NC, _NP, 1)

    ke = x_emb.shape[1]
    wx = W_merge[:ke]
    wp_m = W_merge[ke:ke + _NHID]
    wg_m = W_merge[ke + _NHID:]
    wp = W_ego @ wp_m
    wg = W_glo @ wg_m
    wc = (b_ego @ wp_m).reshape(1, _NHID)
    bias = (b_glo @ wg_m + b_merge).reshape(1, _NHID)
    nw = norm_w.reshape(1, _NHID)
    nb = norm_b.reshape(1, _NHID)

    node_specs = [
        pl.BlockSpec((_BN, ke), lambda i: (i, 0)),
        pl.BlockSpec((_NC, _BN, 16), lambda i: (0, i, 0)),
        pl.BlockSpec((_NC, _BN, 1), lambda i: (0, i, 0)),
        pl.BlockSpec((_BN, 20), lambda i: (i, 0)),
        pl.BlockSpec((_BN, 1), lambda i: (i, 0)),
    ]
    w_specs = [
        pl.BlockSpec((ke, _NHID), lambda i: (0, 0)),
        pl.BlockSpec((16, _NHID), lambda i: (0, 0)),
        pl.BlockSpec((20, _NHID), lambda i: (0, 0)),
        pl.BlockSpec((1, _NHID), lambda i: (0, 0)),
        pl.BlockSpec((1, _NHID), lambda i: (0, 0)),
    ]
    stats_spec = pl.BlockSpec((_G, 3), lambda i: (0, 0))

    stats = pl.pallas_call(
        _pass_a_body,
        grid=(_NBN,),
        in_specs=node_specs + w_specs,
        out_specs=stats_spec,
        out_shape=jax.ShapeDtypeStruct((_G, 3), jnp.float32),
    )(x_emb, parts, cnts3, glo_RWPE, bat, wx, wp, wg, wc, bias)

    out = pl.pallas_call(
        _pass_b_body,
        grid=(_NBN,),
        in_specs=node_specs + w_specs + [
            stats_spec,
            pl.BlockSpec((1, _NHID), lambda i: (0, 0)),
            pl.BlockSpec((1, _NHID), lambda i: (0, 0)),
        ],
        out_specs=pl.BlockSpec((_BN, _NHID), lambda i: (i, 0)),
        out_shape=jax.ShapeDtypeStruct((_N, _NHID), jnp.float32),
    )(x_emb, parts, cnts3, glo_RWPE, bat, wx, wp, wg, wc, bias,
      stats, nw, nb)

    return out
